# baseline (speedup 1.0000x reference)
import jax
import jax.numpy as jnp
from jax.experimental import pallas as pl
from jax.experimental.pallas import tpu as pltpu


def _stage1(x_ref, w_ref, s_ref):
    xt = x_ref[:, :].astype(jnp.bfloat16).T
    w = w_ref[:, :].astype(jnp.bfloat16)
    s_ref[:, :] = jnp.dot(xt, w, preferred_element_type=jnp.float32)


def _make_body(N, F, W, P, OW, TAIL):
    def body(s_ref, b_ref, tail_ref, adj_hbm, out_ref, acc_ref, pbuf, sem):
        i = pl.program_id(0)

        def copy(panel, slot):
            return pltpu.make_async_copy(
                adj_hbm.at[:, pl.ds(panel * W, W)],
                pbuf.at[slot],
                sem.at[slot],
            )

        @pl.when(i == 0)
        def _():
            copy(0, 0).start()

        @pl.when(i + 1 < P)
        def _():
            copy(i + 1, (i + 1) % 2).start()

        slot = i % 2

        @pl.when(i == 0)
        def _():
            copy(0, 0).wait()
            acc_ref[:N, :] = (
                jnp.dot(pbuf[0], s_ref[pl.ds(0, W), :],
                        preferred_element_type=jnp.float32)
                + jnp.dot(tail_ref[:, :TAIL], s_ref[P * W:, :],
                          preferred_element_type=jnp.float32)
            )

        @pl.when((i > 0) & (i < P))
        def _():
            copy(i, slot).wait()
            part = jnp.dot(pbuf[slot], s_ref[pl.ds(i * W, W), :],
                           preferred_element_type=jnp.float32)
            acc_ref[:N, :] += part

        @pl.when(i >= P)
        def _():
            j = i - P
            chunk = acc_ref[pl.ds(j * OW, OW), :]
            out_ref[:, :] = chunk.T + b_ref[:, :]

    return body


def kernel(input, adj, weight, bias):
    C, N = input.shape
    F = weight.shape[1]

    s = pl.pallas_call(
        _stage1,
        in_specs=[
            pl.BlockSpec((C, N), lambda: (0, 0)),
            pl.BlockSpec((C, F), lambda: (0, 0)),
        ],
        out_specs=pl.BlockSpec((N, F), lambda: (0, 0)),
        out_shape=jax.ShapeDtypeStruct((N, F), jnp.float32),
    )(input, weight)

    W = 256
    P = N // W
    TAIL = N - P * W
    LT = (N - TAIL) // 128
    OW = 1024
    NW = pl.cdiv(N, OW)
    G = P + NW

    bias_col = bias.reshape(F, 1)

    def out_map(i):
        return (0, jnp.maximum(i - P, 0))

    out = pl.pallas_call(
        _make_body(N, F, W, P, OW, TAIL),
        grid=(G,),
        in_specs=[
            pl.BlockSpec((N, F), lambda i: (0, 0)),
            pl.BlockSpec((F, 1), lambda i: (0, 0)),
            pl.BlockSpec((N, 128), lambda i: (0, LT)),
            pl.BlockSpec(memory_space=pl.ANY),
        ],
        out_specs=pl.BlockSpec((F, OW), out_map),
        out_shape=jax.ShapeDtypeStruct((F, N), jnp.float32),
        scratch_shapes=[
            pltpu.VMEM((NW * OW, F), jnp.float32),
            pltpu.VMEM((2, N, W), jnp.float32),
            pltpu.SemaphoreType.DMA((2,)),
        ],
    )(s, bias_col, adj, adj)
    return out


# speedup vs baseline: 1.0911x; 1.0911x over previous
import jax
import jax.numpy as jnp
from jax.experimental import pallas as pl
from jax.experimental.pallas import tpu as pltpu


def _make_body(C, N, F, TN, G, STRIPS, KMAIN, TAIL):
    REM = N - (G - 1) * TN

    def body(w_ref, b_ref, tail_ref, x_hbm, adj_hbm, out_ref,
             s_ref, xbuf, pbuf, sem, xsem):
        i = pl.program_id(0)

        def copies(block, slot, rows):
            base = block * TN
            return [
                pltpu.make_async_copy(
                    adj_hbm.at[pl.ds(base, rows), pl.ds(c0, wd)],
                    pbuf.at[slot, pl.ds(0, rows), pl.ds(c0, wd)],
                    sem.at[slot],
                )
                for (c0, wd) in STRIPS
            ]

        def issue(block, slot):
            @pl.when(block < G - 1)
            def _():
                for c in copies(block, slot, TN):
                    c.start()

            @pl.when(block == G - 1)
            def _():
                for c in copies(block, slot, REM):
                    c.start()

        def wait(block, slot):
            @pl.when(block < G - 1)
            def _():
                for c in copies(block, slot, TN):
                    c.wait()

            @pl.when(block == G - 1)
            def _():
                for c in copies(block, slot, REM):
                    c.wait()

        @pl.when(i == 0)
        def _():
            issue(0, 0)
            xcp = pltpu.make_async_copy(x_hbm, xbuf, xsem)
            xcp.start()
            xcp.wait()
            xt = xbuf[:, :].T
            s_ref[:, :] = jnp.dot(xt, w_ref[:, :],
                                  preferred_element_type=jnp.float32)

        @pl.when(i + 1 < G)
        def _():
            issue(i + 1, (i + 1) % 2)

        wait(i, i % 2)

        slot = i % 2
        acc = (
            jnp.dot(pbuf[slot], s_ref[:KMAIN, :],
                    preferred_element_type=jnp.float32)
            + jnp.dot(tail_ref[:, :TAIL], s_ref[KMAIN:, :],
                      preferred_element_type=jnp.float32)
        )
        acc = acc + b_ref[:, :]
        out_ref[:, :] = acc.T

    return body


def kernel(input, adj, weight, bias):
    C, N = input.shape
    F = weight.shape[1]

    TN = 256
    G = pl.cdiv(N, TN)
    KMAIN = (N // 128) * 128
    TAIL = N - KMAIN
    LT = KMAIN // 128
    STRIPS = [(0, 2560), (2560, 2560), (5120, 2560), (7680, 2304)]

    bias2 = bias.reshape(1, F)

    out = pl.pallas_call(
        _make_body(C, N, F, TN, G, STRIPS, KMAIN, TAIL),
        grid=(G,),
        in_specs=[
            pl.BlockSpec((C, F), lambda i: (0, 0)),
            pl.BlockSpec((1, F), lambda i: (0, 0)),
            pl.BlockSpec((TN, 128), lambda i: (i, LT)),
            pl.BlockSpec(memory_space=pl.ANY),
            pl.BlockSpec(memory_space=pl.ANY),
        ],
        out_specs=pl.BlockSpec((F, TN), lambda i: (0, i)),
        out_shape=jax.ShapeDtypeStruct((F, N), jnp.float32),
        scratch_shapes=[
            pltpu.VMEM((N, F), jnp.float32),
            pltpu.VMEM((C, N), jnp.float32),
            pltpu.VMEM((2, TN, KMAIN), jnp.float32),
            pltpu.SemaphoreType.DMA((2,)),
            pltpu.SemaphoreType.DMA,
        ],
    )(weight, bias2, adj, input, adj)
    return out
